# Initial kernel scaffold; baseline (speedup 1.0000x reference)
import jax
import jax.numpy as jnp
from jax.experimental import pallas as pl


def kernel(flows, images):
    raise NotImplementedError("write your pallas kernel here")


# speedup vs baseline: 11.8523x; 11.8523x over previous
import functools

import jax
import jax.numpy as jnp
from jax import lax
from jax.experimental import pallas as pl
from jax.experimental.pallas import tpu as pltpu
from jax.experimental.pallas import tpu_sc as plsc

_B, _L, _C, _H, _W = 2, 24, 8, 64, 64
_HW = _H * _W
_CHW = _C * _HW
_FHW = 2 * _HW
_NSLOT = _B * _L
_NC, _NS = 2, 16
_NW = _NC * _NS
_NGRP = _HW // 16


def _schedule():
    slots = [(b, t) for b in range(_B) for t in range(_L)]
    slots.sort(key=lambda s: -(s[1] + 1))
    loads = [0] * _NW
    jobs = [[] for _ in range(_NW)]
    for (b, t) in slots:
        i = min(range(_NW), key=lambda i: (loads[i], len(jobs[i]), i))
        jobs[i].append((b, t))
        loads[i] += t + 1
    maxj = max(len(j) for j in jobs)
    tbl = []
    for i in range(_NW):
        row = []
        for j in range(maxj):
            if j < len(jobs[i]):
                b, t = jobs[i][j]
                row.append((b * _L + t, b, t))
            else:
                row.append((_NSLOT, 0, -1))
        tbl.append(row)
    return tbl, maxj


_TBL, _MAXJ = _schedule()


def _sel(wid, vals):
    r = jnp.int32(vals[0])
    for i in range(1, _NW):
        r = jnp.where(wid == i, jnp.int32(vals[i]), r)
    return r


def _sc_body(images_hbm, cum_hbm, bxy_hbm, out_hbm,
             img_v, acc_v, ft_v, fk_v, bxy_v):
    cid = lax.axis_index("c")
    sid = lax.axis_index("s")
    wid = sid * _NC + cid

    pltpu.sync_copy(bxy_hbm, bxy_v)

    fW = jnp.float32(_W)
    fH = jnp.float32(_H)

    def group_body(g, _):
        gl = g * 16
        bx = bxy_v[pl.ds(gl, 16)]
        by = bxy_v[pl.ds(_HW + gl, 16)]
        gx = bx + (ft_v[pl.ds(gl, 16)] - fk_v[pl.ds(gl, 16)])
        gy = by + (ft_v[pl.ds(_HW + gl, 16)] - fk_v[pl.ds(_HW + gl, 16)])
        r = lax.rem(gx + 1.0, 2.0)
        r = jnp.where(r < 0.0, r + 2.0, r)
        gxw = r - 1.0
        x = ((gxw + 1.0) * fW - 1.0) * 0.5
        y = ((gy + 1.0) * fH - 1.0) * 0.5
        xi = x.astype(jnp.int32)
        x0 = xi - jnp.where(x < xi.astype(jnp.float32), 1, 0)
        yi = y.astype(jnp.int32)
        y0 = yi - jnp.where(y < yi.astype(jnp.float32), 1, 0)
        wx1 = x - x0.astype(jnp.float32)
        wx0 = 1.0 - wx1
        wy1 = y - y0.astype(jnp.float32)
        wy0 = 1.0 - wy1
        x1 = x0 + 1
        y1 = y0 + 1
        ax0 = jnp.where(x0 >= 0, wx0, 0.0)
        ax1 = jnp.where(x1 <= _W - 1, wx1, 0.0)
        ay0 = jnp.where((y0 >= 0) & (y0 <= _H - 1), wy0, 0.0)
        ay1 = jnp.where((y1 >= 0) & (y1 <= _H - 1), wy1, 0.0)
        x0c = jnp.maximum(x0, 0)
        x1c = jnp.minimum(x1, _W - 1)
        y0c = jnp.minimum(jnp.maximum(y0, 0), _H - 1)
        y1c = jnp.minimum(jnp.maximum(y1, 0), _H - 1)
        row0 = y0c * _W
        row1 = y1c * _W
        a00 = row0 + x0c
        a10 = row0 + x1c
        a01 = row1 + x0c
        a11 = row1 + x1c
        w00 = ax0 * ay0
        w10 = ax1 * ay0
        w01 = ax0 * ay1
        w11 = ax1 * ay1
        for c in range(_C):
            off = c * _HW
            v = plsc.load_gather(img_v, [a00 + off]) * w00
            v = v + plsc.load_gather(img_v, [a10 + off]) * w10
            v = v + plsc.load_gather(img_v, [a01 + off]) * w01
            v = v + plsc.load_gather(img_v, [a11 + off]) * w11
            sl = pl.ds(off + gl, 16)
            acc_v[sl] = acc_v[sl] + v
        return _

    for j in range(_MAXJ):
        slot = _sel(wid, [_TBL[i][j][0] for i in range(_NW)])
        b = _sel(wid, [_TBL[i][j][1] for i in range(_NW)])
        t = _sel(wid, [_TBL[i][j][2] for i in range(_NW)])

        def zero_body16(i, _):
            acc_v[pl.ds(i * 16, 16)] = jnp.zeros((16,), jnp.float32)
            return _

        lax.fori_loop(0, _CHW // 16, zero_body16, 0)

        slotc = jnp.minimum(slot, _NSLOT - 1)
        pltpu.sync_copy(cum_hbm.at[pl.ds(slotc * _FHW, _FHW)], ft_v)

        def k_body(k, _):
            src = b * _L + k
            pltpu.sync_copy(cum_hbm.at[pl.ds(src * _FHW, _FHW)], fk_v)
            pltpu.sync_copy(images_hbm.at[pl.ds(src * _CHW, _CHW)], img_v)
            lax.fori_loop(0, _NGRP, group_body, 0)
            return _

        lax.fori_loop(0, t + 1, k_body, 0)

        pltpu.sync_copy(acc_v, out_hbm.at[pl.ds(slot * _CHW, _CHW)])


_mesh = plsc.VectorSubcoreMesh(core_axis_name="c", subcore_axis_name="s",
                               num_cores=_NC, num_subcores=_NS)

_sc_call = functools.partial(
    pl.kernel,
    mesh=_mesh,
    compiler_params=pltpu.CompilerParams(use_tc_tiling_on_sc=False,
                                         needs_layout_passes=False),
    out_type=jax.ShapeDtypeStruct(((_NSLOT + 1) * _CHW,), jnp.float32),
    scratch_types=[
        pltpu.VMEM((_CHW,), jnp.float32),
        pltpu.VMEM((_CHW,), jnp.float32),
        pltpu.VMEM((_FHW,), jnp.float32),
        pltpu.VMEM((_FHW,), jnp.float32),
        pltpu.VMEM((_FHW,), jnp.float32),
    ],
)(_sc_body)


def kernel(flows, images):
    dtype = flows.dtype
    cum = jnp.cumsum(flows.astype(jnp.float32), axis=1).astype(dtype)
    sy = 2.0 / _H
    sx = 2.0 / _W
    gyc = jnp.linspace(-1.0 + sy * 0.5, 1.0 - sy * 0.5, _H, dtype=dtype)
    gxc = jnp.linspace(-1.0 + sx * 0.5, 1.0 - sx * 0.5, _W, dtype=dtype)
    bx = jnp.tile(gxc, _H)
    by = jnp.repeat(gyc, _W)
    bxy = jnp.concatenate([bx, by])
    out_flat = _sc_call(images.reshape(-1), cum.reshape(-1), bxy)
    return out_flat[: _NSLOT * _CHW].reshape(_B, _L, _C, _H, _W)
